# baseline (speedup 1.0000x reference)
import dataclasses
import functools

import jax
import jax.numpy as jnp
from jax import lax
from jax.experimental import pallas as pl
from jax.experimental.pallas import tpu as pltpu
from jax.experimental.pallas import tpu_sc as plsc

B = 16384
VOCAB = 1000
VP = 1024
EMB = 8
NSLOT = 12
NWIDE = 8
NCLS = 2
NW = 32
BPW = B // NW
L = 16


def _bf(x):
    return x.astype(jnp.bfloat16)


def _prep_body(xw_ref, xd_ref, xden_ref, embp_ref, fcw_ref, fcb_ref,
               t0_ref, t1_ref, idx_ref, dpart_ref):
    embT = _bf(jnp.transpose(embp_ref[...])).astype(jnp.float32)
    for c, t_ref in ((0, t0_ref), (1, t1_ref)):
        rows = []
        for s in range(NSLOT):
            r = None
            for e in range(EMB):
                k = EMB * s + e
                w = _bf(fcw_ref[c : c + 1, k : k + 1]).astype(jnp.float32)
                term = w * embT[e : e + 1, :]
                r = term if r is None else r + term
            rows.append(r)
        t_ref[...] = jnp.concatenate(rows, axis=0)
    xwT = jnp.transpose(xw_ref[...])
    xdT = jnp.transpose(xd_ref[...])
    offw = lax.broadcasted_iota(jnp.int32, (NWIDE, 1), 0) * VP
    offd = (lax.broadcasted_iota(jnp.int32, (NSLOT - NWIDE, 1), 0) + NWIDE) * VP
    idx_ref[...] = jnp.concatenate([xwT + offw, xdT + offd], axis=0)
    xdenT = _bf(jnp.transpose(xden_ref[...])).astype(jnp.float32)
    d = jnp.broadcast_to(fcb_ref[...], (NCLS, B))
    for j in range(4):
        k = NSLOT * EMB + j
        w = _bf(fcw_ref[:, k : k + 1]).astype(jnp.float32)
        d = d + w * xdenT[j : j + 1, :]
    dpart_ref[...] = d


_prep = pl.pallas_call(
    _prep_body,
    out_shape=(
        jax.ShapeDtypeStruct((NSLOT, VP), jnp.float32),
        jax.ShapeDtypeStruct((NSLOT, VP), jnp.float32),
        jax.ShapeDtypeStruct((NSLOT, B), jnp.int32),
        jax.ShapeDtypeStruct((NCLS, B), jnp.float32),
    ),
)


def _sc_compiler_params():
    cp = pltpu.CompilerParams()
    if "needs_layout_passes" in pltpu.CompilerParams.__dataclass_fields__:
        cp = dataclasses.replace(cp, needs_layout_passes=False)
    return cp


@functools.partial(
    pl.kernel,
    out_type=(
        jax.ShapeDtypeStruct((NCLS * B,), jnp.float32),
        jax.ShapeDtypeStruct((NCLS * B,), jnp.float32),
        jax.ShapeDtypeStruct((B,), jnp.int32),
    ),
    mesh=plsc.VectorSubcoreMesh(core_axis_name="c", subcore_axis_name="s"),
    scratch_types=[
        pltpu.VMEM((NSLOT * VP,), jnp.float32),
        pltpu.VMEM((NSLOT * VP,), jnp.float32),
        pltpu.VMEM((NSLOT, BPW), jnp.int32),
        pltpu.VMEM((NCLS, BPW), jnp.float32),
        pltpu.VMEM((NCLS * BPW,), jnp.float32),
        pltpu.VMEM((NCLS * BPW,), jnp.float32),
        pltpu.VMEM((BPW,), jnp.int32),
        pltpu.SemaphoreType.DMA,
        pltpu.SemaphoreType.DMA,
        pltpu.SemaphoreType.DMA,
        pltpu.SemaphoreType.DMA,
    ],
    compiler_params=_sc_compiler_params(),
)
def _sc_main(t0_hbm, t1_hbm, idx_hbm, dpart_hbm,
             lo_hbm, pr_hbm, cl_hbm,
             t0_v, t1_v, idx_v, dp_v, lo_v, pr_v, cl_v,
             sem_t0, sem_t1, sem_i, sem_p):
    wid = lax.axis_index("s") * 2 + lax.axis_index("c")
    base = wid * BPW
    c0t = pltpu.async_copy(t0_hbm, t0_v, sem_t0)
    c1t = pltpu.async_copy(t1_hbm, t1_v, sem_t1)
    ci = pltpu.async_copy(idx_hbm.at[:, pl.ds(base, BPW)], idx_v, sem_i)
    cp = pltpu.async_copy(dpart_hbm.at[:, pl.ds(base, BPW)], dp_v, sem_p)
    ci.wait()
    cp.wait()
    c0t.wait()
    c1t.wait()

    iota2 = 2 * lax.iota(jnp.int32, L)

    @plsc.parallel_loop(0, BPW, step=L, unroll=8)
    def _(c0):
        acc0 = dp_v[0, pl.ds(c0, L)]
        acc1 = dp_v[1, pl.ds(c0, L)]
        for s in range(NSLOT):
            iv = idx_v[s, pl.ds(c0, L)]
            acc0 = acc0 + plsc.load_gather(t0_v, [iv])
            acc1 = acc1 + plsc.load_gather(t1_v, [iv])
        dlt = acc1 - acc0
        t = jnp.exp(-jnp.abs(dlt))
        rs = 1.0 + t
        pw = 1.0 / rs
        pltt = t / rs
        hi = dlt > 0.0
        pos = 2 * c0 + iota2
        plsc.store_scatter(lo_v, [pos], acc0)
        plsc.store_scatter(lo_v, [pos + 1], acc1)
        plsc.store_scatter(pr_v, [pos], jnp.where(hi, pltt, pw))
        plsc.store_scatter(pr_v, [pos + 1], jnp.where(hi, pw, pltt))
        cl_v[pl.ds(c0, L)] = jnp.where(hi, jnp.int32(1), jnp.int32(0))

    pltpu.sync_copy(lo_v, lo_hbm.at[pl.ds(NCLS * base, NCLS * BPW)])
    pltpu.sync_copy(pr_v, pr_hbm.at[pl.ds(NCLS * base, NCLS * BPW)])
    pltpu.sync_copy(cl_v, cl_hbm.at[pl.ds(base, BPW)])


def kernel(x_wide, x_deep, x_dense, emb, fc_w, fc_b):
    emb_pad = jnp.zeros((VP, EMB), jnp.float32).at[:VOCAB].set(emb)
    t0, t1, idxT, dpart = _prep(
        x_wide.astype(jnp.int32), x_deep.astype(jnp.int32), x_dense,
        emb_pad, fc_w, fc_b[:, None])
    lo, pr, cl = _sc_main(t0.reshape(-1), t1.reshape(-1), idxT, dpart)
    return (lo.reshape(B, NCLS), cl.reshape(B, 1), pr.reshape(B, NCLS))


# speedup vs baseline: 1.3336x; 1.3336x over previous
import dataclasses
import functools

import jax
import jax.numpy as jnp
from jax import lax
from jax.experimental import pallas as pl
from jax.experimental.pallas import tpu as pltpu
from jax.experimental.pallas import tpu_sc as plsc

B = 16384
VOCAB = 1000
VP = 1024
EMB = 8
NSLOT = 12
NCLS = 2
NW = 32
BPW = B // NW
L = 16


def _bf(x):
    return x.astype(jnp.bfloat16).astype(jnp.float32)


def _prep_body(embT_ref, a2_ref, dwb_ref, xdT_ref, table_ref, dpart_ref):
    a2 = _bf(a2_ref[...])
    embT = _bf(embT_ref[...])
    t = a2[:, 0:1] * embT[0:1, :]
    for e in range(1, EMB):
        t = t + a2[:, e : e + 1] * embT[e : e + 1, :]
    table_ref[...] = t
    dwb = _bf(dwb_ref[...])
    xdT = _bf(xdT_ref[...])
    d = jnp.broadcast_to(dwb_ref[:, 4:5], (NCLS, B))
    for j in range(4):
        d = d + dwb[:, j : j + 1] * xdT[j : j + 1, :]
    dpart_ref[...] = d


_prep = pl.pallas_call(
    _prep_body,
    out_shape=(
        jax.ShapeDtypeStruct((NSLOT * NCLS, VP), jnp.float32),
        jax.ShapeDtypeStruct((NCLS, B), jnp.float32),
    ),
)


def _sc_compiler_params():
    cp = pltpu.CompilerParams()
    if "needs_layout_passes" in pltpu.CompilerParams.__dataclass_fields__:
        cp = dataclasses.replace(cp, needs_layout_passes=False)
    return cp


@functools.partial(
    pl.kernel,
    out_type=(
        jax.ShapeDtypeStruct((NCLS * B,), jnp.float32),
        jax.ShapeDtypeStruct((NCLS * B,), jnp.float32),
        jax.ShapeDtypeStruct((B,), jnp.int32),
    ),
    mesh=plsc.VectorSubcoreMesh(core_axis_name="c", subcore_axis_name="s"),
    scratch_types=[
        pltpu.VMEM((NSLOT * NCLS * VP,), jnp.float32),
        pltpu.VMEM((NSLOT, BPW), jnp.int32),
        pltpu.VMEM((NCLS, BPW), jnp.float32),
        pltpu.VMEM((NCLS * BPW,), jnp.float32),
        pltpu.VMEM((NCLS * BPW,), jnp.float32),
        pltpu.VMEM((BPW,), jnp.int32),
    ],
    compiler_params=_sc_compiler_params(),
)
def _sc_main(table_hbm, idx_hbm, dpart_hbm, lo_hbm, pr_hbm, cl_hbm,
             table_v, idx_v, dp_v, lo_v, pr_v, cl_v):
    wid = lax.axis_index("s") * 2 + lax.axis_index("c")
    base = wid * BPW
    pltpu.sync_copy(table_hbm, table_v)
    pltpu.sync_copy(idx_hbm.at[:, pl.ds(base, BPW)], idx_v)
    pltpu.sync_copy(dpart_hbm.at[:, pl.ds(base, BPW)], dp_v)

    @pl.loop(0, BPW, step=L)
    def _(c0):
        iv = idx_v[0, pl.ds(c0, L)]
        acc0 = plsc.load_gather(table_v, [iv])
        acc1 = plsc.load_gather(table_v, [iv + VP])
        for s in range(1, NSLOT):
            iv = idx_v[s, pl.ds(c0, L)]
            acc0 = acc0 + plsc.load_gather(table_v, [iv])
            acc1 = acc1 + plsc.load_gather(table_v, [iv + VP])
        acc0 = acc0 + dp_v[0, pl.ds(c0, L)]
        acc1 = acc1 + dp_v[1, pl.ds(c0, L)]
        m = jnp.maximum(acc0, acc1)
        e0 = jnp.exp(acc0 - m)
        e1 = jnp.exp(acc1 - m)
        rs = e0 + e1
        pos = 2 * c0 + 2 * lax.iota(jnp.int32, L)
        plsc.store_scatter(lo_v, [pos], acc0)
        plsc.store_scatter(lo_v, [pos + 1], acc1)
        plsc.store_scatter(pr_v, [pos], e0 / rs)
        plsc.store_scatter(pr_v, [pos + 1], e1 / rs)
        cl_v[pl.ds(c0, L)] = jnp.where(acc1 > acc0, jnp.int32(1), jnp.int32(0))

    pltpu.sync_copy(lo_v, lo_hbm.at[pl.ds(NCLS * base, NCLS * BPW)])
    pltpu.sync_copy(pr_v, pr_hbm.at[pl.ds(NCLS * base, NCLS * BPW)])
    pltpu.sync_copy(cl_v, cl_hbm.at[pl.ds(base, BPW)])


def kernel(x_wide, x_deep, x_dense, emb, fc_w, fc_b):
    x_all = jnp.concatenate(
        [x_wide.astype(jnp.int32), x_deep.astype(jnp.int32)], axis=1)
    idxT = x_all.T + (jnp.arange(NSLOT, dtype=jnp.int32) * (NCLS * VP))[:, None]
    embT = jnp.zeros((EMB, VP), jnp.float32).at[:, :VOCAB].set(emb.T)
    a2 = fc_w[:, : NSLOT * EMB].reshape(NCLS, NSLOT, EMB)
    a2 = a2.transpose(1, 0, 2).reshape(NSLOT * NCLS, EMB)
    dwb = jnp.concatenate([fc_w[:, NSLOT * EMB :], fc_b[:, None]], axis=1)
    table, dpart = _prep(embT, a2, dwb, x_dense.T)
    lo, pr, cl = _sc_main(table.reshape(-1), idxT, dpart)
    return (lo.reshape(B, NCLS), cl.reshape(B, 1), pr.reshape(B, NCLS))
